# baseline (speedup 1.0000x reference)
import functools

import jax
import jax.numpy as jnp
from jax import lax
from jax.experimental import pallas as pl
from jax.experimental.pallas import tpu as pltpu
from jax.experimental.pallas import tpu_sc as plsc

_B = 4096
_S = 200
_D = 64
_L = 16
_NC, _NS = 2, 16
_NW = _NC * _NS
_BPW = _B // _NW
_CH = 32
_NCH = (_S + _CH - 1) // _CH
_SP = _NCH * _CH


def _sc_pool(bx3, lens, table):
    mesh = plsc.VectorSubcoreMesh(core_axis_name="c", subcore_axis_name="s")

    @functools.partial(
        pl.kernel,
        out_type=jax.ShapeDtypeStruct((_B, _D), jnp.float32),
        mesh=mesh,
        compiler_params=pltpu.CompilerParams(use_tc_tiling_on_sc=False),
        scratch_types=[
            pltpu.VMEM((_NCH, _CH), jnp.int32),
            pltpu.VMEM((_NCH, _CH, _D), jnp.float32),
            pltpu.VMEM((_BPW,), jnp.int32),
            pltpu.VMEM((_BPW, _D), jnp.float32),
            pltpu.SemaphoreType.DMA,
        ],
    )
    def k(bx_hbm, lens_hbm, table_hbm, out_hbm, idx_v, rows_v, lens_v,
          outb_v, sem):
        wid = lax.axis_index("s") * _NC + lax.axis_index("c")
        base = wid * _BPW
        pltpu.sync_copy(lens_hbm.at[pl.ds(base, _BPW)], lens_v)

        def group_body(g, carry):
            lv = lens_v[pl.ds(g * _L, _L)]
            for u in range(_L):
                i = g * _L + u
                l = lv[u]
                pltpu.sync_copy(bx_hbm.at[base + i], idx_v)
                cps = [
                    pltpu.async_copy(table_hbm.at[idx_v.at[c]], rows_v.at[c],
                                     sem)
                    for c in range(_NCH)
                ]
                for cp in cps:
                    cp.wait()
                acc = (jnp.zeros((_L,), jnp.float32),) * 4

                def chunk_body(c, acc, l=l):
                    n = jnp.clip(l - c * _CH, 0, _CH)

                    def row_body(r, acc):
                        return tuple(
                            acc[j] + rows_v[c, r, pl.ds(j * _L, _L)]
                            for j in range(4)
                        )

                    return lax.fori_loop(0, n, row_body, acc)

                acc = lax.fori_loop(0, _NCH, chunk_body, acc)
                for j in range(4):
                    outb_v[i, pl.ds(j * _L, _L)] = acc[j]
            return carry

        lax.fori_loop(0, _BPW // _L, group_body, 0)
        pltpu.sync_copy(outb_v, out_hbm.at[pl.ds(base, _BPW)])

    return k(bx3, lens, table)


def _tc_head(doc_sum, lens2, W, b2):
    cat = W.shape[0]

    def body(x_ref, l_ref, w_ref, b_ref, o_ref):
        x = x_ref[...]
        denom = jnp.maximum(l_ref[...].astype(jnp.float32), 1.0)
        doc = x / denom
        z = lax.dot_general(doc, w_ref[...], (((1,), (1,)), ((), ())),
                            preferred_element_type=jnp.float32)
        z = z + b_ref[...]
        m = jnp.max(z, axis=-1, keepdims=True)
        e = jnp.exp(z - m)
        s = jnp.sum(e, axis=-1, keepdims=True)
        o_ref[...] = (z - m) - jnp.log(s)

    return pl.pallas_call(
        body,
        out_shape=jax.ShapeDtypeStruct((_B, cat), jnp.float32),
    )(doc_sum, lens2, W, b2)


def kernel(batch_x, batch_lens, emb_table, W, b):
    bx = batch_x.astype(jnp.int32)
    lens = batch_lens.astype(jnp.int32)
    bx3 = jnp.pad(bx, ((0, 0), (0, _SP - _S))).reshape(_B, _NCH, _CH)
    doc_sum = _sc_pool(bx3, lens, emb_table)
    return _tc_head(doc_sum, lens.reshape(_B, 1), W, b.reshape(1, -1))


# speedup vs baseline: 3.3282x; 3.3282x over previous
import functools

import jax
import jax.numpy as jnp
from jax import lax
from jax.experimental import pallas as pl
from jax.experimental.pallas import tpu as pltpu
from jax.experimental.pallas import tpu_sc as plsc

_B = 4096
_S = 200
_D = 64
_L = 16
_NC, _NS = 2, 16
_NW = _NC * _NS
_BPW = _B // _NW
_CH = 32
_NCH = (_S + _CH - 1) // _CH
_SP = _NCH * _CH
_REG = 136


def _sc_pool(bx3, lens, table):
    mesh = plsc.VectorSubcoreMesh(core_axis_name="c", subcore_axis_name="s")

    @functools.partial(
        pl.kernel,
        out_type=jax.ShapeDtypeStruct((_B, _D), jnp.float32),
        mesh=mesh,
        compiler_params=pltpu.CompilerParams(use_tc_tiling_on_sc=False),
        scratch_types=[
            pltpu.VMEM((2, _NCH, _CH), jnp.int32),
            pltpu.VMEM((2, _NCH, _CH), jnp.int32),
            pltpu.VMEM((2, _NCH, _CH, _D), jnp.float32),
            pltpu.VMEM((_BPW,), jnp.int32),
            pltpu.VMEM((_CH, _D), jnp.float32),
            pltpu.VMEM_SHARED((_NS * _REG, _D), jnp.float32),
            pltpu.SemaphoreType.DMA,
            pltpu.SemaphoreType.DMA,
            pltpu.SemaphoreType.DMA,
        ],
    )
    def k(bx_hbm, lens_hbm, table_hbm, out_hbm, idx_v, sidx_v, rows_v,
          lens_v, zero_v, acc_sh, isem, gsem, ssem):
        cid = lax.axis_index("c")
        sid = lax.axis_index("s")
        wid = sid * _NC + cid
        base = wid * _BPW
        reg0 = sid * _REG

        zeros16 = jnp.zeros((_L,), jnp.float32)
        for r in range(_CH):
            for j in range(4):
                zero_v[r, pl.ds(j * _L, _L)] = zeros16
        for kk in range(4):
            pltpu.sync_copy(zero_v, acc_sh.at[pl.ds(reg0 + kk * _CH, _CH)])
        pltpu.sync_copy(zero_v.at[pl.ds(0, _REG - 4 * _CH)],
                        acc_sh.at[pl.ds(reg0 + 4 * _CH, _REG - 4 * _CH)])

        pltpu.sync_copy(lens_hbm.at[pl.ds(base, _BPW)], lens_v)

        lanes = jax.lax.broadcasted_iota(jnp.int32, (_L,), 0)
        dump = reg0 + _BPW

        def n_chunks(l):
            return (l + _CH - 1) // _CH

        def fire_gathers(p, nc):
            def fg(c, _):
                pltpu.async_copy(table_hbm.at[idx_v.at[p, c]],
                                 rows_v.at[p, c], gsem)
                return 0
            lax.fori_loop(0, nc, fg, 0)

        def wait_gathers(p, nc):
            def wg(c, _):
                pltpu.make_async_copy(table_hbm.at[idx_v.at[p, 0]],
                                      rows_v.at[p, 0], gsem).wait()
                return 0
            lax.fori_loop(0, nc, wg, 0)

        def fire_scatters(p, nc):
            def fs(c, _):
                pltpu.async_copy(rows_v.at[p, c], acc_sh.at[sidx_v.at[p, c]],
                                 ssem, add=True)
                return 0
            lax.fori_loop(0, nc, fs, 0)

        def drain_scatters(p, nc):
            def ds_(c, _):
                pltpu.make_async_copy(rows_v.at[p, 0],
                                      acc_sh.at[sidx_v.at[p, 0]], ssem).wait()
                return 0
            lax.fori_loop(0, nc, ds_, 0)

        def group_body(g, carry):
            lv = lens_v[pl.ds(g * _L, _L)]
            for u in range(_L):
                p = u & 1
                i = g * _L + u
                l = lv[u]
                nc = n_chunks(l)
                if u == 0:
                    pltpu.async_copy(bx_hbm.at[base + i], idx_v.at[p], isem)
                if u >= 2:
                    drain_scatters(p, n_chunks(lv[u - 2]))
                slot = reg0 + i
                for c in range(_NCH):
                    for h in range(2):
                        t = lanes + (c * _CH + h * _L)
                        sidx_v[p, c, pl.ds(h * _L, _L)] = jnp.where(
                            t < l, slot, dump)
                pltpu.make_async_copy(bx_hbm.at[base], idx_v.at[p],
                                      isem).wait()
                fire_gathers(p, nc)
                if u < _L - 1:
                    pltpu.async_copy(bx_hbm.at[base + i + 1],
                                     idx_v.at[p ^ 1], isem)
                wait_gathers(p, nc)
                fire_scatters(p, nc)
            drain_scatters(0, n_chunks(lv[_L - 2]))
            drain_scatters(1, n_chunks(lv[_L - 1]))
            return carry

        lax.fori_loop(0, _BPW // _L, group_body, 0)
        pltpu.sync_copy(acc_sh.at[pl.ds(reg0, _BPW)],
                        out_hbm.at[pl.ds(base, _BPW)])

    return k(bx3, lens, table)


def _tc_head(doc_sum, lens2, W, b2):
    cat = W.shape[0]

    def body(x_ref, l_ref, w_ref, b_ref, o_ref):
        x = x_ref[...]
        denom = jnp.maximum(l_ref[...].astype(jnp.float32), 1.0)
        doc = x / denom
        z = lax.dot_general(doc, w_ref[...], (((1,), (1,)), ((), ())),
                            preferred_element_type=jnp.float32)
        z = z + b_ref[...]
        m = jnp.max(z, axis=-1, keepdims=True)
        e = jnp.exp(z - m)
        s = jnp.sum(e, axis=-1, keepdims=True)
        o_ref[...] = (z - m) - jnp.log(s)

    return pl.pallas_call(
        body,
        out_shape=jax.ShapeDtypeStruct((_B, cat), jnp.float32),
    )(doc_sum, lens2, W, b2)


def kernel(batch_x, batch_lens, emb_table, W, b):
    bx = batch_x.astype(jnp.int32)
    lens = batch_lens.astype(jnp.int32)
    bx3 = jnp.pad(bx, ((0, 0), (0, _SP - _S))).reshape(_B, _NCH, _CH)
    doc_sum = _sc_pool(bx3, lens, emb_table)
    return _tc_head(doc_sum, lens.reshape(_B, 1), W, b.reshape(1, -1))


# speedup vs baseline: 3.3342x; 1.0018x over previous
import functools

import jax
import jax.numpy as jnp
from jax import lax
from jax.experimental import pallas as pl
from jax.experimental.pallas import tpu as pltpu
from jax.experimental.pallas import tpu_sc as plsc

_B = 4096
_S = 200
_D = 64
_L = 16
_NC, _NS = 2, 16
_NW = _NC * _NS
_BPW = _B // _NW
_CH = 32
_NCH = (_S + _CH - 1) // _CH
_SP = _NCH * _CH
_REG = 136


def _sc_pool(bxT, lens, table):
    mesh = plsc.VectorSubcoreMesh(core_axis_name="c", subcore_axis_name="s")

    @functools.partial(
        pl.kernel,
        out_type=jax.ShapeDtypeStruct((_B, _D), jnp.float32),
        mesh=mesh,
        compiler_params=pltpu.CompilerParams(use_tc_tiling_on_sc=False,
                                             needs_layout_passes=False),
        scratch_types=[
            pltpu.VMEM((_S, _BPW), jnp.int32),
            pltpu.VMEM((2, _NCH, _CH), jnp.int32),
            pltpu.VMEM((2, _NCH, _CH), jnp.int32),
            pltpu.VMEM((2, _NCH, _CH, _D), jnp.float32),
            pltpu.VMEM((_BPW,), jnp.int32),
            pltpu.VMEM((_CH, _D), jnp.float32),
            pltpu.VMEM_SHARED((_NS * _REG, _D), jnp.float32),
            pltpu.SemaphoreType.DMA,
            pltpu.SemaphoreType.DMA,
            pltpu.SemaphoreType.DMA,
        ],
    )
    def k(bx_hbm, lens_hbm, table_hbm, out_hbm, blk_v, idx_v, sidx_v, rows_v,
          lens_v, zero_v, acc_sh, isem, gsem, ssem):
        cid = lax.axis_index("c")
        sid = lax.axis_index("s")
        wid = sid * _NC + cid
        base = wid * _BPW
        reg0 = sid * _REG

        pltpu.async_copy(bx_hbm.at[:, pl.ds(base, _BPW)], blk_v, isem)

        zeros16 = jnp.zeros((_L,), jnp.float32)
        for r in range(_CH):
            for j in range(4):
                zero_v[r, pl.ds(j * _L, _L)] = zeros16
        for kk in range(4):
            pltpu.sync_copy(zero_v, acc_sh.at[pl.ds(reg0 + kk * _CH, _CH)])
        pltpu.sync_copy(zero_v.at[pl.ds(0, _REG - 4 * _CH)],
                        acc_sh.at[pl.ds(reg0 + 4 * _CH, _REG - 4 * _CH)])

        pltpu.sync_copy(lens_hbm.at[pl.ds(base, _BPW)], lens_v)
        pltpu.make_async_copy(bx_hbm.at[:, pl.ds(base, _BPW)], blk_v,
                              isem).wait()

        lanes = jax.lax.broadcasted_iota(jnp.int32, (_L,), 0)
        dump = reg0 + _BPW

        def n_chunks(l):
            return (l + _CH - 1) // _CH

        def fire_gathers(p, nc):
            def fg(c, _):
                pltpu.async_copy(table_hbm.at[idx_v.at[p, c]],
                                 rows_v.at[p, c], gsem)
                return 0
            lax.fori_loop(0, nc, fg, 0)

        def wait_gathers(p, nc):
            def wg(c, _):
                pltpu.make_async_copy(table_hbm.at[idx_v.at[p, 0]],
                                      rows_v.at[p, 0], gsem).wait()
                return 0
            lax.fori_loop(0, nc, wg, 0)

        def fire_scatters(p, nc):
            def fs(c, _):
                pltpu.async_copy(rows_v.at[p, c], acc_sh.at[sidx_v.at[p, c]],
                                 ssem, add=True)
                return 0
            lax.fori_loop(0, nc, fs, 0)

        def drain_scatters(p, nc):
            def ds_(c, _):
                pltpu.make_async_copy(rows_v.at[p, 0],
                                      acc_sh.at[sidx_v.at[p, 0]], ssem).wait()
                return 0
            lax.fori_loop(0, nc, ds_, 0)

        def group_body(g, carry):
            lv = lens_v[pl.ds(g * _L, _L)]
            for u in range(_L):
                p = u & 1
                i = g * _L + u
                l = lv[u]
                nc = n_chunks(l)
                if u >= 2:
                    drain_scatters(p, n_chunks(lv[u - 2]))
                slot = reg0 + i
                col = jnp.full((_L,), i, jnp.int32)
                for c in range(_NCH):
                    for h in range(2):
                        t = lanes + (c * _CH + h * _L)
                        trow = jnp.minimum(t, _S - 1)
                        vals = plsc.load_gather(blk_v, [trow, col])
                        idx_v[p, c, pl.ds(h * _L, _L)] = vals
                        sidx_v[p, c, pl.ds(h * _L, _L)] = jnp.where(
                            t < l, slot, dump)
                fire_gathers(p, nc)
                wait_gathers(p, nc)
                fire_scatters(p, nc)
            drain_scatters(0, n_chunks(lv[_L - 2]))
            drain_scatters(1, n_chunks(lv[_L - 1]))
            return carry

        lax.fori_loop(0, _BPW // _L, group_body, 0)
        pltpu.sync_copy(acc_sh.at[pl.ds(reg0, _BPW)],
                        out_hbm.at[pl.ds(base, _BPW)])

    return k(bxT, lens, table)


def _tc_head(doc_sum, lens2, W, b2):
    cat = W.shape[0]

    def body(x_ref, l_ref, w_ref, b_ref, o_ref):
        x = x_ref[...]
        denom = jnp.maximum(l_ref[...].astype(jnp.float32), 1.0)
        doc = x / denom
        z = lax.dot_general(doc, w_ref[...], (((1,), (1,)), ((), ())),
                            preferred_element_type=jnp.float32)
        z = z + b_ref[...]
        m = jnp.max(z, axis=-1, keepdims=True)
        e = jnp.exp(z - m)
        s = jnp.sum(e, axis=-1, keepdims=True)
        o_ref[...] = (z - m) - jnp.log(s)

    return pl.pallas_call(
        body,
        out_shape=jax.ShapeDtypeStruct((_B, cat), jnp.float32),
    )(doc_sum, lens2, W, b2)


def kernel(batch_x, batch_lens, emb_table, W, b):
    bx = batch_x.astype(jnp.int32)
    lens = batch_lens.astype(jnp.int32)
    doc_sum = _sc_pool(bx.T, lens, emb_table)
    return _tc_head(doc_sum, lens.reshape(_B, 1), W, b.reshape(1, -1))
